# baseline (speedup 1.0000x reference)
import jax
import jax.numpy as jnp
from jax import lax
from jax.experimental import pallas as pl
from jax.experimental.pallas import tpu as pltpu
from jax.experimental.pallas import tpu_sc as plsc

NV = 8192
DV = 4
DC = 8
NE = NV * DV
BATCH = 64
W = 32
NIT = 5
NT = 16
EPT = NE // NT
VPT = NV // NT
MC = 512
NMC = EPT // MC
GPC = MC // 128
HC = 128
NHC = VPT // HC
L = 16

_mesh = plsc.VectorSubcoreMesh(core_axis_name="c", subcore_axis_name="s")
_f32 = jnp.float32
_bf16 = jnp.bfloat16
_PK = plsc.PackFormat.INTERLEAVED
_SC_PARAMS = pltpu.CompilerParams(use_tc_tiling_on_sc=False,
                                  needs_layout_passes=False)


def _minsum_mc(t_v, el_v, p, mbase, first):
    def g_body(g, carry):
        r0 = g * DC
        q = []
        for j in range(DC):
            x = t_v[p, r0 + j, :]
            if not first:
                x = x - el_v[mbase + r0 + j, :]
            q.append(x)
        a = [jnp.abs(x) for x in q]
        pre = [a[0]]
        for j in range(1, DC - 1):
            pre.append(jnp.minimum(pre[-1], a[j]))
        suf_rev = [a[DC - 1]]
        for j in range(DC - 2, 0, -1):
            suf_rev.append(jnp.minimum(suf_rev[-1], a[j]))
        qb = [x < 0.0 for x in q]
        tot = qb[0]
        for j in range(1, DC):
            tot = tot ^ qb[j]
        for j in range(DC):
            if j == 0:
                m = suf_rev[DC - 2]
            elif j == DC - 1:
                m = pre[DC - 2]
            else:
                m = jnp.minimum(pre[j - 1], suf_rev[DC - 2 - j])
            el_v[mbase + r0 + j, :] = jnp.where(tot ^ qb[j], -m, m)
        return carry

    lax.fori_loop(0, MC // DC, g_body, 0)


def _body(chan_hbm, varc_hbm, etvc_hbm, out_hbm, e_hbm, llr_hbm,
          idxa_v, idxb_v, big_v, el_v, chan_v, o2_v, lb2_v, ob_v,
          sa0, sa1, sa2, sw, sb0, sb1, swb0, swb1):
    cid = lax.axis_index("c")
    sid = lax.axis_index("s")
    ebase = cid * NE + sid * EPT
    vbase = cid * NV + sid * VPT

    sa = (sa0, sa1, sa2)
    sb = (sb0, sb1)
    swb = (swb0, swb1)

    pltpu.sync_copy(varc_hbm.at[cid, sid], idxa_v)
    pltpu.sync_copy(etvc_hbm.at[cid, sid], idxb_v)

    pltpu.sync_copy(chan_hbm.at[pl.ds(vbase, VPT)], chan_v)
    for h in range(NHC):

        def i_body(r, carry):
            a = chan_v[h * HC + r, pl.ds(0, L)]
            b = chan_v[h * HC + r, pl.ds(L, L)]
            lb2_v[0, r, :] = plsc.pack(a, b, format=_PK)
            return carry

        lax.fori_loop(0, HC, i_body, 0)
        pltpu.sync_copy(lb2_v.at[0], llr_hbm.at[pl.ds(vbase + h * HC, HC)])
    plsc.subcore_barrier()

    for it in range(NIT):
        first = it == 0

        def fire_a(m):
            p = m % 3
            return [pltpu.async_copy(
                llr_hbm.at[idxa_v.at[m * GPC + q]],
                big_v.at[p, pl.ds(q * 128, 128)], sa[p])
                for q in range(GPC)]

        pend = {0: fire_a(0), 1: fire_a(1)}
        wbs = []
        for m in range(NMC):
            p = m % 3
            cur = pend.pop(m)
            if m + 2 < NMC:
                pend[m + 2] = fire_a(m + 2)
            for cp in cur:
                cp.wait()
            _minsum_mc(big_v, el_v, p, m * MC, first)
            wbs.append(pltpu.async_copy(
                el_v.at[pl.ds(m * MC, MC)],
                e_hbm.at[pl.ds(ebase + m * MC, MC)], sw))
        for cp in wbs:
            cp.wait()
        plsc.subcore_barrier()

        def fire_b(h):
            pg = h % 3
            return [pltpu.async_copy(e_hbm.at[idxb_v.at[d, h]],
                                     big_v.at[pg, pl.ds(d * HC, HC)], sa[pg])
                    for d in range(DV)]

        pend = {0: fire_b(0), 1: fire_b(1)}
        wbs = [None, None]
        for h in range(NHC):
            pg = h % 3
            po = h % 2
            cur = pend.pop(h)
            if h + 2 < NHC:
                pend[h + 2] = fire_b(h + 2)
            for cp in cur:
                cp.wait()
            if wbs[po] is not None:
                for cp in wbs[po]:
                    cp.wait()
                wbs[po] = None

            def r_body(r, carry):
                sa_ = chan_v[h * HC + r, pl.ds(0, L)]
                sb_ = chan_v[h * HC + r, pl.ds(L, L)]
                for d in range(DV):
                    ea, eb = plsc.unpack(big_v[pg, d * HC + r, :], format=_PK)
                    sa_ = sa_ + ea
                    sb_ = sb_ + eb
                lb2_v[po, r, :] = plsc.pack(sa_, sb_, format=_PK)
                o2_v[po, r, pl.ds(0, L)] = 1.0 / (1.0 + jnp.exp(sa_))
                o2_v[po, r, pl.ds(L, L)] = 1.0 / (1.0 + jnp.exp(sb_))
                return carry

            lax.fori_loop(0, HC, r_body, 0)

            def t_body(k, carry):
                colk = jnp.full((L,), 0, jnp.int32) + k
                for half in range(HC // L):
                    rows = lax.iota(jnp.int32, L) + half * L
                    vals = plsc.load_gather(o2_v.at[po], [rows, colk])
                    ob_v[k, pl.ds(h * HC + half * L, L)] = vals
                return carry

            lax.fori_loop(0, W, t_body, 0)
            vb = pl.ds(vbase + h * HC, HC)
            wb1 = pltpu.async_copy(lb2_v.at[po], llr_hbm.at[vb], swb[po])
            wbs[po] = [wb1]
        owb = pltpu.async_copy(
            ob_v,
            out_hbm.at[it, pl.ds(cid * W, W), pl.ds(vbase - cid * NV, VPT)],
            sb0)
        for p in range(2):
            if wbs[p] is not None:
                for cp in wbs[p]:
                    cp.wait()
        owb.wait()
        plsc.subcore_barrier()


_K = pl.kernel(
    _body,
    out_type=(
        jax.ShapeDtypeStruct((NIT, BATCH, NV), _f32),
        jax.ShapeDtypeStruct((2 * NE, W), _bf16),
        jax.ShapeDtypeStruct((2 * NV, W), _bf16),
    ),
    mesh=_mesh,
    scratch_types=[
        pltpu.VMEM((NMC * GPC, 128), jnp.int32),
        pltpu.VMEM((DV, NHC, 128), jnp.int32),
        pltpu.VMEM((3, MC, W), _bf16),
        pltpu.VMEM((EPT, W), _bf16),
        pltpu.VMEM((VPT, W), _f32),
        pltpu.VMEM((2, HC, W), _f32),
        pltpu.VMEM((2, HC, W), _bf16),
        pltpu.VMEM((W, VPT), _f32),
    ] + [pltpu.SemaphoreType.DMA] * 8,
    compiler_params=_SC_PARAMS,
)


def kernel(channelLLR, edgeToVar, edgeToVarMask, oddToEven, edgeToChk):
    chanT = (channelLLR.T.astype(_f32)
             .reshape(NV, 2, W).transpose(1, 0, 2).reshape(2 * NV, W))
    var = oddToEven.astype(jnp.int32)
    varc = jnp.stack([var, var + NV]).reshape(2, NT, NMC * GPC, 128)
    etv = edgeToVar.astype(jnp.int32).T
    etvc = (jnp.stack([etv, etv + NE])
            .reshape(2, DV, NT, NHC, 128).transpose(0, 2, 1, 3, 4))
    out, _, _ = _K(chanT, varc, etvc)
    return out
